# baseline (speedup 1.0000x reference)
import functools

import jax
import jax.numpy as jnp
from jax import lax
from jax.experimental import pallas as pl
from jax.experimental.pallas import tpu as pltpu
from jax.experimental.pallas import tpu_sc as plsc

NC = 2
NS = 16
L = 16
NW = NC * NS

N_NODES = 10000
N_EDGES = 320000
NP = 10240
F = 128
K = 80
EW = N_EDGES // NW
KE = 80
CH = 125
CH0 = 126
CH1 = 2 * CH - CH0
IR = 6
RB = 4

_MESH = plsc.VectorSubcoreMesh(
    core_axis_name="c", subcore_axis_name="s", num_cores=NC, num_subcores=NS)


@functools.partial(
    pl.kernel,
    out_type=jax.ShapeDtypeStruct((4, NP), jnp.float32),
    mesh=_MESH,
    scratch_types=[
        pltpu.VMEM((EW // KE, 1, KE), jnp.int32),
        pltpu.VMEM((EW // KE, 1, KE), jnp.int32),
        pltpu.SemaphoreType.DMA,
        pltpu.VMEM((NP,), jnp.float32),
        pltpu.VMEM((NP,), jnp.float32),
        pltpu.VMEM((NS, 1, NP // NS), jnp.float32),
        pltpu.VMEM((NP // NS,), jnp.float32),
        pltpu.VMEM_SHARED((2, NS, 1, NP), jnp.float32),
    ],
    compiler_params=pltpu.CompilerParams(needs_layout_passes=False),
)
def _deg_kernel(ei_hbm, out_hbm, idx_v, idx_w, dsem, h_out, h_in, rbuf,
                rres, stage):
    cid = lax.axis_index("c")
    sid = lax.axis_index("s")
    wid = sid * NC + cid
    base = wid * EW
    ones = jnp.ones((L,), jnp.float32)
    zeros = jnp.zeros((L,), jnp.float32)

    def _zero(i, _):
        for k in range(8):
            h_out[pl.ds((i * 8 + k) * L, L)] = zeros
            h_in[pl.ds((i * 8 + k) * L, L)] = zeros
        return _
    lax.fori_loop(0, NP // (8 * L), _zero, None)

    cw = EW // KE
    dcp = pltpu.async_copy(ei_hbm.at[1, pl.ds(wid * cw, cw), :, :], idx_w,
                           dsem)
    pltpu.sync_copy(ei_hbm.at[0, pl.ds(wid * cw, cw), :, :], idx_v)
    def _hist_out(i, _):
        for k in range(KE // L):
            plsc.addupdate_scatter(
                h_out, [idx_v[i, 0, pl.ds(k * L, L)]], ones)
        return _
    lax.fori_loop(0, cw, _hist_out, None)

    dcp.wait()
    def _hist_in(i, _):
        for k in range(KE // L):
            plsc.addupdate_scatter(
                h_in, [idx_w[i, 0, pl.ds(k * L, L)]], ones)
        return _
    lax.fori_loop(0, cw, _hist_in, None)

    pltpu.sync_copy(h_out, stage.at[0, sid, 0, :])
    pltpu.sync_copy(h_in, stage.at[1, sid, 0, :])
    plsc.subcore_barrier()

    nc_ = NP // NS
    col0 = sid * nc_
    for r in range(2):
        pltpu.sync_copy(stage.at[r, :, pl.ds(0, 1), pl.ds(col0, nc_)], rbuf)
        def _red(i, _):
            acc = rbuf[0, 0, pl.ds(i * L, L)]
            for t in range(1, NS):
                acc = acc + rbuf[t, 0, pl.ds(i * L, L)]
            rres[pl.ds(i * L, L)] = acc
            return _
        lax.fori_loop(0, nc_ // L, _red, None)
        pltpu.sync_copy(rres, out_hbm.at[2 * cid + r, pl.ds(col0, nc_)])


def _scale_mm_body(x_ref, d_ref, w_ref, o_ref):
    d = jnp.transpose(d_ref[...])
    s = lax.rsqrt(jnp.maximum(d[:, 0:1] + d[:, 2:3], 1.0))
    o_ref[...] = jnp.dot(
        x_ref[...].astype(jnp.bfloat16) * s.astype(jnp.bfloat16),
        w_ref[...], preferred_element_type=jnp.float32)


@functools.partial(
    pl.kernel,
    out_type=jax.ShapeDtypeStruct((NC, NP, F), jnp.float32),
    mesh=_MESH,
    scratch_types=[
        pltpu.VMEM((IR, 2, KE), jnp.int32),
        pltpu.VMEM((RB, KE, F), jnp.float32),
        pltpu.VMEM_SHARED((NP, F), jnp.float32),
        pltpu.SemaphoreType.DMA((IR,)),
        pltpu.SemaphoreType.DMA((RB,)),
        pltpu.SemaphoreType.DMA((RB,)),
    ],
)
def _edge_kernel(z_hbm, ei_hbm, out_hbm, idx_r, rows_v, acc_sh,
                 isem, gsem, ssem):
    cid = lax.axis_index("c")
    sid = lax.axis_index("s")
    base_c = jnp.where(cid == 0, sid * CH0, NS * CH0 + sid * CH1)
    nch = jnp.where(cid == 0, CH0, CH1)

    def _fill(i, _):
        r = i // (F // L)
        c = lax.rem(i, F // L)
        rows_v[0, r, pl.ds(c * L, L)] = jnp.zeros((L,), jnp.float32)
        return _
    lax.fori_loop(0, KE * (F // L), _fill, None)
    row0 = sid * (NP // NS)
    def _zero(k, _):
        pltpu.async_copy(rows_v.at[0], acc_sh.at[pl.ds(row0 + k * KE, KE), :],
                         gsem.at[0])
        return _
    lax.fori_loop(0, (NP // NS) // KE, _zero, None)
    def _zwait(k, _):
        pltpu.make_async_copy(rows_v.at[0],
                              acc_sh.at[pl.ds(row0 + k * KE, KE), :],
                              gsem.at[0]).wait()
        return _
    lax.fori_loop(0, (NP // NS) // KE, _zwait, None)
    plsc.subcore_barrier()

    def _idx_load(c):
        pltpu.async_copy(ei_hbm.at[0, base_c + c, 0], idx_r.at[lax.rem(c, IR), 0],
                         isem.at[lax.rem(c, IR)])
        pltpu.async_copy(ei_hbm.at[1, base_c + c, 0], idx_r.at[lax.rem(c, IR), 1],
                         isem.at[lax.rem(c, IR)])

    def _idx_wait(c):
        pltpu.make_async_copy(ei_hbm.at[0, base_c + c, 0],
                              idx_r.at[lax.rem(c, IR), 0],
                              isem.at[lax.rem(c, IR)]).wait()
        pltpu.make_async_copy(ei_hbm.at[1, base_c + c, 0],
                              idx_r.at[lax.rem(c, IR), 1],
                              isem.at[lax.rem(c, IR)]).wait()

    def _gather(c, p):
        pltpu.async_copy(z_hbm.at[idx_r.at[lax.rem(c, IR), 0]], rows_v.at[p],
                         gsem.at[p])

    def _gather_wait(c, p):
        pltpu.make_async_copy(z_hbm.at[idx_r.at[lax.rem(c, IR), 0]],
                              rows_v.at[p], gsem.at[p]).wait()

    def _scatter(c, p):
        pltpu.async_copy(rows_v.at[p], acc_sh.at[idx_r.at[lax.rem(c, IR), 1]],
                         ssem.at[p], add=True)

    def _scatter_wait(c, p):
        pltpu.make_async_copy(rows_v.at[p],
                              acc_sh.at[idx_r.at[lax.rem(c, IR), 1]],
                              ssem.at[p]).wait()

    for c in range(IR - 1):
        _idx_load(jnp.int32(c))
    for c in range(RB - 1):
        _idx_wait(jnp.int32(c))
        _gather(jnp.int32(c), c)

    def _body(j, _):
        p = lax.rem(j, RB)

        @pl.when(j > 0)
        def _():
            _scatter_wait(j - 1, lax.rem(j - 1, RB))

        @pl.when(j + IR - 1 < nch)
        def _():
            _idx_load(j + IR - 1)

        @pl.when(j + RB - 1 < nch)
        def _():
            _idx_wait(j + RB - 1)
            _gather(j + RB - 1, lax.rem(j + RB - 1, RB))

        _gather_wait(j, p)
        _scatter(j, p)
        return _
    lax.fori_loop(0, nch, _body, None)
    _scatter_wait(nch - 1, lax.rem(nch - 1, RB))
    plsc.subcore_barrier()

    nr = NP // NS
    pltpu.sync_copy(acc_sh.at[pl.ds(row0, nr), :],
                    out_hbm.at[cid, pl.ds(row0, nr), :])


def _final_body(p_ref, d_ref, bc_ref, wl_ref, bl_ref, o_ref, acc_ref):
    i = pl.program_id(0)
    nb = pl.num_programs(0)

    @pl.when(i == 0)
    def _():
        acc_ref[...] = jnp.zeros_like(acc_ref)

    blk = p_ref.shape[1]
    p = p_ref[0] + p_ref[1]
    d = jnp.transpose(d_ref[...])
    s = lax.rsqrt(jnp.maximum(d[:, 1:2] + d[:, 3:4], 1.0))
    h = jnp.maximum(p * s + bc_ref[...], 0.0)
    rows = i * blk + lax.broadcasted_iota(jnp.int32, (blk, 1), 0)
    h = jnp.where(rows < N_NODES, h, 0.0)
    acc_ref[...] += jnp.sum(h, axis=0, keepdims=True)

    @pl.when(i == nb - 1)
    def _():
        m = acc_ref[...] / float(N_NODES)
        logits = lax.dot_general(m, wl_ref[...], (((1,), (1,)), ((), ())),
                                 preferred_element_type=jnp.float32)
        logits = logits + bl_ref[...]
        e = jnp.exp(logits - jnp.max(logits))
        o_ref[...] = e / jnp.sum(e)


def kernel(in_feat, edge_index, W_conv, b_conv, W_lin, b_lin):
    ei = edge_index.astype(jnp.int32)

    ei4 = ei.reshape(2, -1, 1, KE)
    deg = _deg_kernel(ei4)

    z = pl.pallas_call(
        _scale_mm_body,
        grid=(5,),
        in_specs=[
            pl.BlockSpec((2048, F), lambda i: (i, 0)),
            pl.BlockSpec((4, 2048), lambda i: (0, i)),
            pl.BlockSpec((F, F), lambda i: (0, 0)),
        ],
        out_specs=pl.BlockSpec((2048, F), lambda i: (i, 0)),
        out_shape=jax.ShapeDtypeStruct((N_NODES, F), jnp.float32),
    )(in_feat, deg, W_conv.astype(jnp.bfloat16))

    partials = _edge_kernel(z, ei4)

    BLK = 2048
    p = pl.pallas_call(
        _final_body,
        grid=(NP // BLK,),
        in_specs=[
            pl.BlockSpec((NC, BLK, F), lambda i: (0, i, 0)),
            pl.BlockSpec((4, BLK), lambda i: (0, i)),
            pl.BlockSpec((1, F), lambda i: (0, 0)),
            pl.BlockSpec(W_lin.shape, lambda i: (0, 0)),
            pl.BlockSpec((1, W_lin.shape[0]), lambda i: (0, 0)),
        ],
        out_specs=pl.BlockSpec((1, W_lin.shape[0]), lambda i: (0, 0)),
        out_shape=jax.ShapeDtypeStruct((1, W_lin.shape[0]), jnp.float32),
        scratch_shapes=[pltpu.VMEM((1, F), jnp.float32)],
    )(partials, deg, b_conv.reshape(1, F), W_lin,
      b_lin.reshape(1, W_lin.shape[0]))

    return p.reshape(W_lin.shape[0])
